# baseline (speedup 1.0000x reference)
import jax
import jax.numpy as jnp
from jax import lax
from jax.experimental import pallas as pl

_N = 16384
_C = 1000
_BR = 512


def _onehot_block(x_ref, o_ref):
    col = lax.broadcasted_iota(jnp.int32, (_BR, _C), 1)
    o_ref[...] = (x_ref[...] == col).astype(jnp.int32)


def kernel(x):
    x2 = x.reshape(_N, 1)
    return pl.pallas_call(
        _onehot_block,
        grid=(_N // _BR,),
        in_specs=[pl.BlockSpec((_BR, 1), lambda i: (i, 0))],
        out_specs=pl.BlockSpec((_BR, _C), lambda i: (i, 0)),
        out_shape=jax.ShapeDtypeStruct((_N, _C), jnp.int32),
    )(x2)


# speedup vs baseline: 1.1389x; 1.1389x over previous
import jax
import jax.numpy as jnp
from jax import lax
from jax.experimental import pallas as pl

_N = 16384
_C = 1000
_BR = 512


def _onehot_block(x_ref, o_ref):
    col = lax.broadcasted_iota(jnp.int32, (_BR, _C), 1)
    xv = x_ref[0, 0, :].reshape(_BR, 1)
    o_ref[...] = (xv == col).astype(jnp.int32)


def kernel(x):
    x3 = x.reshape(_N // _BR, 1, _BR)
    return pl.pallas_call(
        _onehot_block,
        grid=(_N // _BR,),
        in_specs=[pl.BlockSpec((1, 1, _BR), lambda i: (i, 0, 0))],
        out_specs=pl.BlockSpec((_BR, _C), lambda i: (i, 0)),
        out_shape=jax.ShapeDtypeStruct((_N, _C), jnp.int32),
    )(x3)
